# baseline (speedup 1.0000x reference)
import functools

import jax
import jax.numpy as jnp
from jax import lax
from jax.experimental import pallas as pl
from jax.experimental.pallas import tpu as pltpu
from jax.experimental.pallas import tpu_sc as plsc

B, S, D = 4, 4096, 1024
P, L = 1024, 16
TOP_K = 8
BLK = 128
NSTEPS = S // BLK
OUT_S = TOP_K * L + S
NC, NS = 2, 16
NW = NC * NS


def _k1_body(x_ref, key_ref, out_ref, sim_ref, idx_ref, acc_ref, knorm_ref):
    j = pl.program_id(0)

    xb = x_ref[...]
    out_ref[...] = xb

    part = jnp.sum(xb, axis=1)

    @pl.when(j == 0)
    def _init():
        acc_ref[...] = part
        k = key_ref[...]
        ksq = jnp.sum(k * k, axis=1, keepdims=True)
        knorm_ref[...] = k * lax.rsqrt(jnp.maximum(ksq, 1e-12))

    @pl.when(j > 0)
    def _accum():
        acc_ref[...] += part

    @pl.when(j == NSTEPS - 1)
    def _finish():
        mean = acc_ref[...] * (1.0 / S)
        msq = jnp.sum(mean * mean, axis=1, keepdims=True)
        xn = mean * lax.rsqrt(jnp.maximum(msq, 1e-12))
        sim = lax.dot_general(
            xn, knorm_ref[...],
            dimension_numbers=(((1,), (1,)), ((), ())),
            preferred_element_type=jnp.float32,
        )
        sim_ref[...] = sim

        iota = lax.broadcasted_iota(jnp.int32, (B, P), 1)
        work = sim
        cols = []
        for _ in range(TOP_K):
            m = jnp.max(work, axis=1, keepdims=True)
            cand = jnp.where(work == m, iota, P)
            sel = jnp.min(cand, axis=1, keepdims=True)
            cols.append(sel)
            work = jnp.where(iota == sel, -1e30, work)
        idx_ref[...] = jnp.concatenate(cols, axis=1)


def _k1(x_embed, prompt_key):
    return pl.pallas_call(
        _k1_body,
        grid=(NSTEPS,),
        in_specs=[
            pl.BlockSpec((B, BLK, D), lambda j: (0, j, 0)),
            pl.BlockSpec((P, D), lambda j: (0, 0)),
        ],
        out_specs=[
            pl.BlockSpec((B, BLK, D), lambda j: (0, j + 1, 0)),
            pl.BlockSpec((B, P), lambda j: (0, 0)),
            pl.BlockSpec((B, TOP_K), lambda j: (0, 0)),
        ],
        out_shape=[
            jax.ShapeDtypeStruct((B, OUT_S, D), jnp.float32),
            jax.ShapeDtypeStruct((B, P), jnp.float32),
            jax.ShapeDtypeStruct((B, TOP_K), jnp.int32),
        ],
        scratch_shapes=[
            pltpu.VMEM((B, D), jnp.float32),
            pltpu.VMEM((P, D), jnp.float32),
        ],
    )(x_embed, prompt_key)


def _k2_body(idx_hbm, table_hbm, out_hbm, ids_v, rows_v, sem):
    wid = lax.axis_index("s") * NC + lax.axis_index("c")

    pltpu.sync_copy(idx_hbm, ids_v)
    lane = lax.broadcasted_iota(jnp.int32, (16,), 0)
    my_idx = plsc.load_gather(ids_v, [jnp.full((16,), wid, jnp.int32)])
    row_ids = my_idx * L + lane

    pltpu.async_copy(table_hbm.at[row_ids], rows_v, sem).wait()
    pltpu.sync_copy(rows_v, out_hbm.at[pl.ds(wid * L, L)])


@functools.cache
def _k2():
    return pl.kernel(
        _k2_body,
        out_type=jax.ShapeDtypeStruct((B * TOP_K * L, D), jnp.float32),
        mesh=plsc.VectorSubcoreMesh(core_axis_name="c", subcore_axis_name="s"),
        compiler_params=pltpu.CompilerParams(needs_layout_passes=False),
        scratch_types=[
            pltpu.VMEM((NW,), jnp.int32),
            pltpu.VMEM((L, D), jnp.float32),
            pltpu.SemaphoreType.DMA,
        ],
    )


def _kg_body(idx_ref, prompt_ref, prompted_in, out_ref, sem):
    del prompted_in
    copies = []
    for i in range(B * TOP_K):
        b, k = divmod(i, TOP_K)
        c = pltpu.make_async_copy(
            prompt_ref.at[idx_ref[i]],
            out_ref.at[b, pl.ds(k * L, L), :],
            sem,
        )
        c.start()
        copies.append(c)
    for c in copies:
        c.wait()


def _kg(idx_flat, prompt, prompted):
    grid_spec = pltpu.PrefetchScalarGridSpec(
        num_scalar_prefetch=1,
        grid=(1,),
        in_specs=[
            pl.BlockSpec(memory_space=pl.ANY),
            pl.BlockSpec(memory_space=pl.ANY),
        ],
        out_specs=pl.BlockSpec(memory_space=pl.ANY),
        scratch_shapes=[pltpu.SemaphoreType.DMA],
    )
    return pl.pallas_call(
        _kg_body,
        grid_spec=grid_spec,
        out_shape=jax.ShapeDtypeStruct((B, OUT_S, D), jnp.float32),
        input_output_aliases={2: 0},
    )(idx_flat, prompt, prompted)


def _k23_body(idx_ref, prompt_ref, prompted_in, out_ref):
    del idx_ref, prompted_in
    out_ref[...] = prompt_ref[...]


def _k23(idx_flat, prompt, prompted):
    grid_spec = pltpu.PrefetchScalarGridSpec(
        num_scalar_prefetch=1,
        grid=(B * TOP_K,),
        in_specs=[
            pl.BlockSpec((1, L, D), lambda i, idx: (idx[i], 0, 0)),
            pl.BlockSpec(memory_space=pl.ANY),
        ],
        out_specs=pl.BlockSpec((1, L, D), lambda i, idx: (i // TOP_K, i % TOP_K, 0)),
    )
    return pl.pallas_call(
        _k23_body,
        grid_spec=grid_spec,
        out_shape=jax.ShapeDtypeStruct((B, OUT_S, D), jnp.float32),
        input_output_aliases={2: 0},
    )(idx_flat, prompt, prompted)


def _k3_body(g_ref, prompted_in, out_ref):
    del prompted_in
    out_ref[...] = g_ref[...]


def _k3(g, prompted):
    return pl.pallas_call(
        _k3_body,
        grid=(1,),
        in_specs=[
            pl.BlockSpec((B, TOP_K * L, D), lambda j: (0, 0, 0)),
            pl.BlockSpec(memory_space=pl.ANY),
        ],
        out_specs=pl.BlockSpec((B, TOP_K * L, D), lambda j: (0, 0, 0)),
        out_shape=jax.ShapeDtypeStruct((B, OUT_S, D), jnp.float32),
        input_output_aliases={1: 0},
    )(g, prompted)


def kernel(x_embed, prompt, prompt_key):
    prompted, similarity, idx = _k1(x_embed, prompt_key)
    prompted = _kg(idx.reshape(B * TOP_K), prompt, prompted)
    return prompted, similarity, idx


# speedup vs baseline: 2.0606x; 2.0606x over previous
import functools

import jax
import jax.numpy as jnp
from jax import lax
from jax.experimental import pallas as pl
from jax.experimental.pallas import tpu as pltpu
from jax.experimental.pallas import tpu_sc as plsc

B, S, D = 4, 4096, 1024
P, L = 1024, 16
TOP_K = 8
BLK = 128
NSTEPS = S // BLK
OUT_S = TOP_K * L + S
NC, NS = 2, 16
NW = NC * NS


def _k1_body(x_ref, key_ref, out_ref, sim_ref, idx_ref, acc_ref, knorm_ref):
    j = pl.program_id(0)

    xb = x_ref[...]
    out_ref[...] = xb

    part = jnp.sum(xb, axis=1)

    @pl.when(j == 0)
    def _init():
        acc_ref[...] = part
        k = key_ref[...]
        ksq = jnp.sum(k * k, axis=1, keepdims=True)
        knorm_ref[...] = k * lax.rsqrt(jnp.maximum(ksq, 1e-12))

    @pl.when(j > 0)
    def _accum():
        acc_ref[...] += part

    @pl.when(j == NSTEPS - 1)
    def _finish():
        mean = acc_ref[...] * (1.0 / S)
        msq = jnp.sum(mean * mean, axis=1, keepdims=True)
        xn = mean * lax.rsqrt(jnp.maximum(msq, 1e-12))
        sim = lax.dot_general(
            xn, knorm_ref[...],
            dimension_numbers=(((1,), (1,)), ((), ())),
            preferred_element_type=jnp.float32,
        )
        sim_ref[...] = sim

        iota = lax.broadcasted_iota(jnp.int32, (B, P), 1)
        work = sim
        cols = []
        for _ in range(TOP_K):
            m = jnp.max(work, axis=1, keepdims=True)
            cand = jnp.where(work == m, iota, P)
            sel = jnp.min(cand, axis=1, keepdims=True)
            cols.append(sel)
            work = jnp.where(iota == sel, -1e30, work)
        idx_ref[...] = jnp.concatenate(cols, axis=1)


def _k1(x_embed, prompt_key):
    return pl.pallas_call(
        _k1_body,
        grid=(NSTEPS,),
        in_specs=[
            pl.BlockSpec((B, BLK, D), lambda j: (0, j, 0)),
            pl.BlockSpec((P, D), lambda j: (0, 0)),
        ],
        out_specs=[
            pl.BlockSpec((B, BLK, D), lambda j: (0, j + 1, 0)),
            pl.BlockSpec((B, P), lambda j: (0, 0)),
            pl.BlockSpec((B, TOP_K), lambda j: (0, 0)),
        ],
        out_shape=[
            jax.ShapeDtypeStruct((B, OUT_S, D), jnp.float32),
            jax.ShapeDtypeStruct((B, P), jnp.float32),
            jax.ShapeDtypeStruct((B, TOP_K), jnp.int32),
        ],
        scratch_shapes=[
            pltpu.VMEM((B, D), jnp.float32),
            pltpu.VMEM((P, D), jnp.float32),
        ],
    )(x_embed, prompt_key)


def _k2_body(idx_hbm, table_hbm, out_hbm, ids_v, rows_v, sem):
    wid = lax.axis_index("s") * NC + lax.axis_index("c")

    pltpu.sync_copy(idx_hbm, ids_v)
    lane = lax.broadcasted_iota(jnp.int32, (16,), 0)
    my_idx = plsc.load_gather(ids_v, [jnp.full((16,), wid, jnp.int32)])
    row_ids = my_idx * L + lane

    pltpu.async_copy(table_hbm.at[row_ids], rows_v, sem).wait()
    pltpu.sync_copy(rows_v, out_hbm.at[pl.ds(wid * L, L)])


@functools.cache
def _k2():
    return pl.kernel(
        _k2_body,
        out_type=jax.ShapeDtypeStruct((B * TOP_K * L, D), jnp.float32),
        mesh=plsc.VectorSubcoreMesh(core_axis_name="c", subcore_axis_name="s"),
        compiler_params=pltpu.CompilerParams(needs_layout_passes=False),
        scratch_types=[
            pltpu.VMEM((NW,), jnp.int32),
            pltpu.VMEM((L, D), jnp.float32),
            pltpu.SemaphoreType.DMA,
        ],
    )


def _kg_body(idx_ref, prompt_ref, prompted_in, out_ref, gbuf, sem):
    del prompted_in
    copies = []
    for i in range(B * TOP_K):
        b, k = divmod(i, TOP_K)
        c = pltpu.make_async_copy(
            prompt_ref.at[idx_ref[i]],
            gbuf.at[b, pl.ds(k * L, L), :],
            sem,
        )
        c.start()
        copies.append(c)
    for c in copies:
        c.wait()
    out_ref[...] = gbuf[...]


def _kg(idx_flat, prompt, prompted):
    grid_spec = pltpu.PrefetchScalarGridSpec(
        num_scalar_prefetch=1,
        grid=(1,),
        in_specs=[
            pl.BlockSpec(memory_space=pl.ANY),
            pl.BlockSpec(memory_space=pl.ANY),
        ],
        out_specs=pl.BlockSpec((B, TOP_K * L, D), lambda i, idx: (0, 0, 0)),
        scratch_shapes=[
            pltpu.VMEM((B, TOP_K * L, D), jnp.float32),
            pltpu.SemaphoreType.DMA,
        ],
    )
    return pl.pallas_call(
        _kg_body,
        grid_spec=grid_spec,
        out_shape=jax.ShapeDtypeStruct((B, OUT_S, D), jnp.float32),
        input_output_aliases={2: 0},
    )(idx_flat, prompt, prompted)


def _k23_body(idx_ref, prompt_ref, prompted_in, out_ref):
    del idx_ref, prompted_in
    out_ref[...] = prompt_ref[...]


def _k23(idx_flat, prompt, prompted):
    grid_spec = pltpu.PrefetchScalarGridSpec(
        num_scalar_prefetch=1,
        grid=(B * TOP_K,),
        in_specs=[
            pl.BlockSpec((1, L, D), lambda i, idx: (idx[i], 0, 0)),
            pl.BlockSpec(memory_space=pl.ANY),
        ],
        out_specs=pl.BlockSpec((1, L, D), lambda i, idx: (i // TOP_K, i % TOP_K, 0)),
    )
    return pl.pallas_call(
        _k23_body,
        grid_spec=grid_spec,
        out_shape=jax.ShapeDtypeStruct((B, OUT_S, D), jnp.float32),
        input_output_aliases={2: 0},
    )(idx_flat, prompt, prompted)


def _k3_body(g_ref, prompted_in, out_ref):
    del prompted_in
    out_ref[...] = g_ref[...]


def _k3(g, prompted):
    return pl.pallas_call(
        _k3_body,
        grid=(1,),
        in_specs=[
            pl.BlockSpec((B, TOP_K * L, D), lambda j: (0, 0, 0)),
            pl.BlockSpec(memory_space=pl.ANY),
        ],
        out_specs=pl.BlockSpec((B, TOP_K * L, D), lambda j: (0, 0, 0)),
        out_shape=jax.ShapeDtypeStruct((B, OUT_S, D), jnp.float32),
        input_output_aliases={1: 0},
    )(g, prompted)


def kernel(x_embed, prompt, prompt_key):
    prompted, similarity, idx = _k1(x_embed, prompt_key)
    prompted = _kg(idx.reshape(B * TOP_K), prompt, prompted)
    return prompted, similarity, idx


# speedup vs baseline: 2.2132x; 1.0741x over previous
import functools

import jax
import jax.numpy as jnp
from jax import lax
from jax.experimental import pallas as pl
from jax.experimental.pallas import tpu as pltpu
from jax.experimental.pallas import tpu_sc as plsc

B, S, D = 4, 4096, 1024
P, L = 1024, 16
TOP_K = 8
PL = TOP_K * L
OUT_S = PL + S
XBLK = 384
XSTEPS = OUT_S // XBLK
NC, NS = 2, 16
NW = NC * NS


def _k1_body(x_ref, key_ref, out_ref, sim_ref, idx_ref, acc_ref, knorm_ref,
             carry_ref):
    j = pl.program_id(0)

    xb = x_ref[...]
    out_ref[:, 0:PL, :] = carry_ref[...]
    out_ref[:, PL:XBLK, :] = xb[:, 0 : XBLK - PL, :]
    carry_ref[...] = xb[:, XBLK - PL : XBLK, :]

    @pl.when(j == 0)
    def _init():
        acc_ref[...] = jnp.sum(xb, axis=1)
        k = key_ref[...]
        ksq = jnp.sum(k * k, axis=1, keepdims=True)
        knorm_ref[...] = k * lax.rsqrt(jnp.maximum(ksq, 1e-12))

    @pl.when((j > 0) & (j < XSTEPS - 1))
    def _accum():
        acc_ref[...] += jnp.sum(xb, axis=1)

    @pl.when(j == XSTEPS - 1)
    def _accum_tail():
        acc_ref[...] += jnp.sum(xb[:, 0 : XBLK - PL, :], axis=1)

    @pl.when(j == XSTEPS - 1)
    def _finish():
        mean = acc_ref[...] * (1.0 / S)
        msq = jnp.sum(mean * mean, axis=1, keepdims=True)
        xn = mean * lax.rsqrt(jnp.maximum(msq, 1e-12))
        sim = lax.dot_general(
            xn, knorm_ref[...],
            dimension_numbers=(((1,), (1,)), ((), ())),
            preferred_element_type=jnp.float32,
        )
        sim_ref[...] = sim

        iota = lax.broadcasted_iota(jnp.int32, (B, P), 1)
        work = sim
        cols = []
        for _ in range(TOP_K):
            m = jnp.max(work, axis=1, keepdims=True)
            cand = jnp.where(work == m, iota, P)
            sel = jnp.min(cand, axis=1, keepdims=True)
            cols.append(sel)
            work = jnp.where(iota == sel, -1e30, work)
        idx_ref[...] = jnp.concatenate(cols, axis=1)


def _k1(x_embed, prompt_key):
    return pl.pallas_call(
        _k1_body,
        grid=(XSTEPS,),
        in_specs=[
            pl.BlockSpec((B, XBLK, D), lambda j: (0, j, 0)),
            pl.BlockSpec((P, D), lambda j: (0, 0)),
        ],
        out_specs=[
            pl.BlockSpec((B, XBLK, D), lambda j: (0, j, 0)),
            pl.BlockSpec((B, P), lambda j: (0, 0)),
            pl.BlockSpec((B, TOP_K), lambda j: (0, 0)),
        ],
        out_shape=[
            jax.ShapeDtypeStruct((B, OUT_S, D), jnp.float32),
            jax.ShapeDtypeStruct((B, P), jnp.float32),
            jax.ShapeDtypeStruct((B, TOP_K), jnp.int32),
        ],
        scratch_shapes=[
            pltpu.VMEM((B, D), jnp.float32),
            pltpu.VMEM((P, D), jnp.float32),
            pltpu.VMEM((B, PL, D), jnp.float32),
        ],
    )(x_embed, prompt_key)


def _k2_body(idx_hbm, table_hbm, out_hbm, ids_v, rows_v, sem):
    wid = lax.axis_index("s") * NC + lax.axis_index("c")

    pltpu.sync_copy(idx_hbm, ids_v)
    lane = lax.broadcasted_iota(jnp.int32, (16,), 0)
    my_idx = plsc.load_gather(ids_v, [jnp.full((16,), wid, jnp.int32)])
    row_ids = my_idx * L + lane

    pltpu.async_copy(table_hbm.at[row_ids], rows_v, sem).wait()
    pltpu.sync_copy(rows_v, out_hbm.at[pl.ds(wid * L, L)])


@functools.cache
def _k2():
    return pl.kernel(
        _k2_body,
        out_type=jax.ShapeDtypeStruct((B * TOP_K * L, D), jnp.float32),
        mesh=plsc.VectorSubcoreMesh(core_axis_name="c", subcore_axis_name="s"),
        compiler_params=pltpu.CompilerParams(needs_layout_passes=False),
        scratch_types=[
            pltpu.VMEM((NW,), jnp.int32),
            pltpu.VMEM((L, D), jnp.float32),
            pltpu.SemaphoreType.DMA,
        ],
    )


def _kg_body(idx_ref, prompt_ref, prompted_in, out_ref, gbuf, sem):
    del prompted_in
    copies = []
    for i in range(B * TOP_K):
        b, k = divmod(i, TOP_K)
        c = pltpu.make_async_copy(
            prompt_ref.at[idx_ref[i]],
            gbuf.at[b, pl.ds(k * L, L), :],
            sem,
        )
        c.start()
        copies.append(c)
    for c in copies:
        c.wait()
    out_ref[...] = gbuf[...]


def _kg(idx_flat, prompt, prompted):
    grid_spec = pltpu.PrefetchScalarGridSpec(
        num_scalar_prefetch=1,
        grid=(1,),
        in_specs=[
            pl.BlockSpec(memory_space=pl.ANY),
            pl.BlockSpec(memory_space=pl.ANY),
        ],
        out_specs=pl.BlockSpec((B, TOP_K * L, D), lambda i, idx: (0, 0, 0)),
        scratch_shapes=[
            pltpu.VMEM((B, TOP_K * L, D), jnp.float32),
            pltpu.SemaphoreType.DMA,
        ],
    )
    return pl.pallas_call(
        _kg_body,
        grid_spec=grid_spec,
        out_shape=jax.ShapeDtypeStruct((B, OUT_S, D), jnp.float32),
        input_output_aliases={2: 0},
    )(idx_flat, prompt, prompted)


def _k23_body(idx_ref, prompt_ref, prompted_in, out_ref):
    del idx_ref, prompted_in
    out_ref[...] = prompt_ref[...]


def _k23(idx_flat, prompt, prompted):
    grid_spec = pltpu.PrefetchScalarGridSpec(
        num_scalar_prefetch=1,
        grid=(B * TOP_K,),
        in_specs=[
            pl.BlockSpec((1, L, D), lambda i, idx: (idx[i], 0, 0)),
            pl.BlockSpec(memory_space=pl.ANY),
        ],
        out_specs=pl.BlockSpec((1, L, D), lambda i, idx: (i // TOP_K, i % TOP_K, 0)),
    )
    return pl.pallas_call(
        _k23_body,
        grid_spec=grid_spec,
        out_shape=jax.ShapeDtypeStruct((B, OUT_S, D), jnp.float32),
        input_output_aliases={2: 0},
    )(idx_flat, prompt, prompted)


def _k3_body(g_ref, prompted_in, out_ref):
    del prompted_in
    out_ref[...] = g_ref[...]


def _k3(g, prompted):
    return pl.pallas_call(
        _k3_body,
        grid=(1,),
        in_specs=[
            pl.BlockSpec((B, TOP_K * L, D), lambda j: (0, 0, 0)),
            pl.BlockSpec(memory_space=pl.ANY),
        ],
        out_specs=pl.BlockSpec((B, TOP_K * L, D), lambda j: (0, 0, 0)),
        out_shape=jax.ShapeDtypeStruct((B, OUT_S, D), jnp.float32),
        input_output_aliases={1: 0},
    )(g, prompted)


def kernel(x_embed, prompt, prompt_key):
    prompted, similarity, idx = _k1(x_embed, prompt_key)
    prompted = _kg(idx.reshape(B * TOP_K), prompt, prompted)
    return prompted, similarity, idx


# speedup vs baseline: 2.2196x; 1.0029x over previous
import functools

import jax
import jax.numpy as jnp
from jax import lax
from jax.experimental import pallas as pl
from jax.experimental.pallas import tpu as pltpu
from jax.experimental.pallas import tpu_sc as plsc

B, S, D = 4, 4096, 1024
P, L = 1024, 16
TOP_K = 8
PL = TOP_K * L
OUT_S = PL + S
XBLK = 384
XSTEPS = OUT_S // XBLK
NC, NS = 2, 16
NW = NC * NS


def _k1_body(x_ref, key_ref, out_ref, sim_ref, idx_ref, acc_ref, knorm_ref,
             carry_ref):
    j = pl.program_id(0)

    xb = x_ref[...]
    out_ref[:, 0:PL, :] = carry_ref[...]
    out_ref[:, PL:XBLK, :] = xb[:, 0 : XBLK - PL, :]
    carry_ref[...] = xb[:, XBLK - PL : XBLK, :]

    @pl.when(j == 0)
    def _init():
        acc_ref[...] = jnp.sum(xb, axis=1)
        k = key_ref[...]
        ksq = jnp.sum(k * k, axis=1, keepdims=True)
        knorm_ref[...] = k * lax.rsqrt(jnp.maximum(ksq, 1e-12))

    @pl.when((j > 0) & (j < XSTEPS - 1))
    def _accum():
        acc_ref[...] += jnp.sum(xb, axis=1)

    @pl.when(j == XSTEPS - 1)
    def _accum_tail():
        acc_ref[...] += jnp.sum(xb[:, 0 : XBLK - PL, :], axis=1)

    @pl.when(j == XSTEPS - 1)
    def _finish():
        mean = acc_ref[...] * (1.0 / S)
        msq = jnp.sum(mean * mean, axis=1, keepdims=True)
        xn = mean * lax.rsqrt(jnp.maximum(msq, 1e-12))
        sim = lax.dot_general(
            xn, knorm_ref[...],
            dimension_numbers=(((1,), (1,)), ((), ())),
            preferred_element_type=jnp.float32,
        )
        sim_ref[...] = sim

        iota = lax.broadcasted_iota(jnp.int32, (B, P), 1)
        work = sim
        cols = []
        for _ in range(TOP_K):
            m = jnp.max(work, axis=1, keepdims=True)
            cand = jnp.where(work == m, iota, P)
            sel = jnp.min(cand, axis=1, keepdims=True)
            cols.append(sel)
            work = jnp.where(iota == sel, -1e30, work)
        idx_ref[...] = jnp.concatenate(cols, axis=1)


def _k1(x_embed, prompt_key):
    return pl.pallas_call(
        _k1_body,
        grid=(XSTEPS,),
        in_specs=[
            pl.BlockSpec((B, XBLK, D), lambda j: (0, j, 0)),
            pl.BlockSpec((P, D), lambda j: (0, 0)),
        ],
        out_specs=[
            pl.BlockSpec((B, XBLK, D), lambda j: (0, j, 0)),
            pl.BlockSpec((B, P), lambda j: (0, 0)),
            pl.BlockSpec((B, TOP_K), lambda j: (0, 0)),
        ],
        out_shape=[
            jax.ShapeDtypeStruct((B, OUT_S, D), jnp.float32),
            jax.ShapeDtypeStruct((B, P), jnp.float32),
            jax.ShapeDtypeStruct((B, TOP_K), jnp.int32),
        ],
        scratch_shapes=[
            pltpu.VMEM((B, D), jnp.float32),
            pltpu.VMEM((P, D), jnp.float32),
            pltpu.VMEM((B, PL, D), jnp.float32),
        ],
    )(x_embed, prompt_key)


def _k2_body(idx_hbm, table_hbm, out_hbm, ids_v, rows_v, sem):
    wid = lax.axis_index("s") * NC + lax.axis_index("c")

    pltpu.sync_copy(idx_hbm, ids_v)
    lane = lax.broadcasted_iota(jnp.int32, (16,), 0)
    my_idx = plsc.load_gather(ids_v, [jnp.full((16,), wid, jnp.int32)])
    row_ids = my_idx * L + lane

    pltpu.async_copy(table_hbm.at[row_ids], rows_v, sem).wait()
    pltpu.sync_copy(rows_v, out_hbm.at[pl.ds(wid * L, L)])


@functools.cache
def _k2():
    return pl.kernel(
        _k2_body,
        out_type=jax.ShapeDtypeStruct((B * TOP_K * L, D), jnp.float32),
        mesh=plsc.VectorSubcoreMesh(core_axis_name="c", subcore_axis_name="s"),
        compiler_params=pltpu.CompilerParams(needs_layout_passes=False),
        scratch_types=[
            pltpu.VMEM((NW,), jnp.int32),
            pltpu.VMEM((L, D), jnp.float32),
            pltpu.SemaphoreType.DMA,
        ],
    )


def _kg_body(idx_ref, prompt_ref, prompted_in, out_ref, sem):
    del prompted_in
    copies = []
    for i in range(B * TOP_K):
        b, k = divmod(i, TOP_K)
        c = pltpu.make_async_copy(
            prompt_ref.at[idx_ref[i]],
            out_ref.at[b, pl.ds(k * L, L), :],
            sem,
        )
        c.start()
        copies.append(c)
    for c in copies:
        c.wait()


def _kg(idx_flat, prompt, prompted):
    grid_spec = pltpu.PrefetchScalarGridSpec(
        num_scalar_prefetch=1,
        grid=(1,),
        in_specs=[
            pl.BlockSpec(memory_space=pl.ANY),
            pl.BlockSpec(memory_space=pl.ANY),
        ],
        out_specs=pl.BlockSpec((B, TOP_K * L, D), lambda i, idx: (0, 0, 0)),
        scratch_shapes=[
            pltpu.SemaphoreType.DMA,
        ],
    )
    return pl.pallas_call(
        _kg_body,
        grid_spec=grid_spec,
        out_shape=jax.ShapeDtypeStruct((B, OUT_S, D), jnp.float32),
        input_output_aliases={2: 0},
    )(idx_flat, prompt, prompted)


def _k23_body(idx_ref, prompt_ref, prompted_in, out_ref):
    del idx_ref, prompted_in
    out_ref[...] = prompt_ref[...]


def _k23(idx_flat, prompt, prompted):
    grid_spec = pltpu.PrefetchScalarGridSpec(
        num_scalar_prefetch=1,
        grid=(B * TOP_K,),
        in_specs=[
            pl.BlockSpec((1, L, D), lambda i, idx: (idx[i], 0, 0)),
            pl.BlockSpec(memory_space=pl.ANY),
        ],
        out_specs=pl.BlockSpec((1, L, D), lambda i, idx: (i // TOP_K, i % TOP_K, 0)),
    )
    return pl.pallas_call(
        _k23_body,
        grid_spec=grid_spec,
        out_shape=jax.ShapeDtypeStruct((B, OUT_S, D), jnp.float32),
        input_output_aliases={2: 0},
    )(idx_flat, prompt, prompted)


def _k3_body(g_ref, prompted_in, out_ref):
    del prompted_in
    out_ref[...] = g_ref[...]


def _k3(g, prompted):
    return pl.pallas_call(
        _k3_body,
        grid=(1,),
        in_specs=[
            pl.BlockSpec((B, TOP_K * L, D), lambda j: (0, 0, 0)),
            pl.BlockSpec(memory_space=pl.ANY),
        ],
        out_specs=pl.BlockSpec((B, TOP_K * L, D), lambda j: (0, 0, 0)),
        out_shape=jax.ShapeDtypeStruct((B, OUT_S, D), jnp.float32),
        input_output_aliases={1: 0},
    )(g, prompted)


def kernel(x_embed, prompt, prompt_key):
    prompted, similarity, idx = _k1(x_embed, prompt_key)
    prompted = _kg(idx.reshape(B * TOP_K), prompt, prompted)
    return prompted, similarity, idx


# speedup vs baseline: 2.2271x; 1.0034x over previous
import functools

import jax
import jax.numpy as jnp
from jax import lax
from jax.experimental import pallas as pl
from jax.experimental.pallas import tpu as pltpu
from jax.experimental.pallas import tpu_sc as plsc

B, S, D = 4, 4096, 1024
P, L = 1024, 16
TOP_K = 8
PL = TOP_K * L
OUT_S = PL + S
XBLK = 528
XSTEPS = OUT_S // XBLK
NC, NS = 2, 16
NW = NC * NS


def _k1_body(x_ref, key_ref, out_ref, sim_ref, idx_ref, acc_ref, knorm_ref,
             carry_ref):
    j = pl.program_id(0)

    xb = x_ref[...]
    out_ref[:, 0:PL, :] = carry_ref[...]
    out_ref[:, PL:XBLK, :] = xb[:, 0 : XBLK - PL, :]
    carry_ref[...] = xb[:, XBLK - PL : XBLK, :]

    @pl.when(j == 0)
    def _init():
        acc_ref[...] = jnp.sum(xb, axis=1)
        k = key_ref[...]
        ksq = jnp.sum(k * k, axis=1, keepdims=True)
        knorm_ref[...] = k * lax.rsqrt(jnp.maximum(ksq, 1e-12))

    @pl.when((j > 0) & (j < XSTEPS - 1))
    def _accum():
        acc_ref[...] += jnp.sum(xb, axis=1)

    @pl.when(j == XSTEPS - 1)
    def _accum_tail():
        acc_ref[...] += jnp.sum(xb[:, 0 : XBLK - PL, :], axis=1)

    @pl.when(j == XSTEPS - 1)
    def _finish():
        mean = acc_ref[...] * (1.0 / S)
        msq = jnp.sum(mean * mean, axis=1, keepdims=True)
        xn = mean * lax.rsqrt(jnp.maximum(msq, 1e-12))
        sim = lax.dot_general(
            xn, knorm_ref[...],
            dimension_numbers=(((1,), (1,)), ((), ())),
            preferred_element_type=jnp.float32,
        )
        sim_ref[...] = sim

        iota = lax.broadcasted_iota(jnp.int32, (B, P), 1)
        work = sim
        cols = []
        for _ in range(TOP_K):
            m = jnp.max(work, axis=1, keepdims=True)
            cand = jnp.where(work == m, iota, P)
            sel = jnp.min(cand, axis=1, keepdims=True)
            cols.append(sel)
            work = jnp.where(iota == sel, -1e30, work)
        idx_ref[...] = jnp.concatenate(cols, axis=1)


def _k1(x_embed, prompt_key):
    return pl.pallas_call(
        _k1_body,
        grid=(XSTEPS,),
        in_specs=[
            pl.BlockSpec((B, XBLK, D), lambda j: (0, j, 0)),
            pl.BlockSpec((P, D), lambda j: (0, 0)),
        ],
        out_specs=[
            pl.BlockSpec((B, XBLK, D), lambda j: (0, j, 0)),
            pl.BlockSpec((B, P), lambda j: (0, 0)),
            pl.BlockSpec((B, TOP_K), lambda j: (0, 0)),
        ],
        out_shape=[
            jax.ShapeDtypeStruct((B, OUT_S, D), jnp.float32),
            jax.ShapeDtypeStruct((B, P), jnp.float32),
            jax.ShapeDtypeStruct((B, TOP_K), jnp.int32),
        ],
        scratch_shapes=[
            pltpu.VMEM((B, D), jnp.float32),
            pltpu.VMEM((P, D), jnp.float32),
            pltpu.VMEM((B, PL, D), jnp.float32),
        ],
    )(x_embed, prompt_key)


def _k2_body(idx_hbm, table_hbm, out_hbm, ids_v, rows_v, sem):
    wid = lax.axis_index("s") * NC + lax.axis_index("c")

    pltpu.sync_copy(idx_hbm, ids_v)
    lane = lax.broadcasted_iota(jnp.int32, (16,), 0)
    my_idx = plsc.load_gather(ids_v, [jnp.full((16,), wid, jnp.int32)])
    row_ids = my_idx * L + lane

    pltpu.async_copy(table_hbm.at[row_ids], rows_v, sem).wait()
    pltpu.sync_copy(rows_v, out_hbm.at[pl.ds(wid * L, L)])


@functools.cache
def _k2():
    return pl.kernel(
        _k2_body,
        out_type=jax.ShapeDtypeStruct((B * TOP_K * L, D), jnp.float32),
        mesh=plsc.VectorSubcoreMesh(core_axis_name="c", subcore_axis_name="s"),
        compiler_params=pltpu.CompilerParams(needs_layout_passes=False),
        scratch_types=[
            pltpu.VMEM((NW,), jnp.int32),
            pltpu.VMEM((L, D), jnp.float32),
            pltpu.SemaphoreType.DMA,
        ],
    )


def _kg_body(idx_ref, prompt_ref, prompted_in, out_ref, sem):
    del prompted_in
    copies = []
    for i in range(B * TOP_K):
        b, k = divmod(i, TOP_K)
        c = pltpu.make_async_copy(
            prompt_ref.at[idx_ref[i]],
            out_ref.at[b, pl.ds(k * L, L), :],
            sem,
        )
        c.start()
        copies.append(c)
    for c in copies:
        c.wait()


def _kg(idx_flat, prompt, prompted):
    grid_spec = pltpu.PrefetchScalarGridSpec(
        num_scalar_prefetch=1,
        grid=(1,),
        in_specs=[
            pl.BlockSpec(memory_space=pl.ANY),
            pl.BlockSpec(memory_space=pl.ANY),
        ],
        out_specs=pl.BlockSpec((B, TOP_K * L, D), lambda i, idx: (0, 0, 0)),
        scratch_shapes=[
            pltpu.SemaphoreType.DMA,
        ],
    )
    return pl.pallas_call(
        _kg_body,
        grid_spec=grid_spec,
        out_shape=jax.ShapeDtypeStruct((B, OUT_S, D), jnp.float32),
        input_output_aliases={2: 0},
    )(idx_flat, prompt, prompted)


def _k23_body(idx_ref, prompt_ref, prompted_in, out_ref):
    del idx_ref, prompted_in
    out_ref[...] = prompt_ref[...]


def _k23(idx_flat, prompt, prompted):
    grid_spec = pltpu.PrefetchScalarGridSpec(
        num_scalar_prefetch=1,
        grid=(B * TOP_K,),
        in_specs=[
            pl.BlockSpec((1, L, D), lambda i, idx: (idx[i], 0, 0)),
            pl.BlockSpec(memory_space=pl.ANY),
        ],
        out_specs=pl.BlockSpec((1, L, D), lambda i, idx: (i // TOP_K, i % TOP_K, 0)),
    )
    return pl.pallas_call(
        _k23_body,
        grid_spec=grid_spec,
        out_shape=jax.ShapeDtypeStruct((B, OUT_S, D), jnp.float32),
        input_output_aliases={2: 0},
    )(idx_flat, prompt, prompted)


def _k3_body(g_ref, prompted_in, out_ref):
    del prompted_in
    out_ref[...] = g_ref[...]


def _k3(g, prompted):
    return pl.pallas_call(
        _k3_body,
        grid=(1,),
        in_specs=[
            pl.BlockSpec((B, TOP_K * L, D), lambda j: (0, 0, 0)),
            pl.BlockSpec(memory_space=pl.ANY),
        ],
        out_specs=pl.BlockSpec((B, TOP_K * L, D), lambda j: (0, 0, 0)),
        out_shape=jax.ShapeDtypeStruct((B, OUT_S, D), jnp.float32),
        input_output_aliases={1: 0},
    )(g, prompted)


def kernel(x_embed, prompt, prompt_key):
    prompted, similarity, idx = _k1(x_embed, prompt_key)
    prompted = _kg(idx.reshape(B * TOP_K), prompt, prompted)
    return prompted, similarity, idx


# speedup vs baseline: 2.2334x; 1.0028x over previous
import jax
import jax.numpy as jnp
from jax import lax
from jax.experimental import pallas as pl
from jax.experimental.pallas import tpu as pltpu

B, S, D = 4, 4096, 1024
P, L = 1024, 16
TOP_K = 8
PL = TOP_K * L
OUT_S = PL + S
XBLK = 528
XSTEPS = OUT_S // XBLK


def _k1_body(x_ref, key_ref, out_ref, sim_ref, idx_ref, acc_ref, knorm_ref,
             carry_ref):
    j = pl.program_id(0)

    xb = x_ref[...]
    out_ref[:, 0:PL, :] = carry_ref[...]
    out_ref[:, PL:XBLK, :] = xb[:, 0 : XBLK - PL, :]
    carry_ref[...] = xb[:, XBLK - PL : XBLK, :]

    @pl.when(j == 0)
    def _init():
        acc_ref[...] = jnp.sum(xb, axis=1)
        k = key_ref[...]
        ksq = jnp.sum(k * k, axis=1, keepdims=True)
        knorm_ref[...] = k * lax.rsqrt(jnp.maximum(ksq, 1e-12))

    @pl.when((j > 0) & (j < XSTEPS - 1))
    def _accum():
        acc_ref[...] += jnp.sum(xb, axis=1)

    @pl.when(j == XSTEPS - 1)
    def _accum_tail():
        acc_ref[...] += jnp.sum(xb[:, 0 : XBLK - PL, :], axis=1)

    @pl.when(j == XSTEPS - 1)
    def _finish():
        mean = acc_ref[...] * (1.0 / S)
        msq = jnp.sum(mean * mean, axis=1, keepdims=True)
        xn = mean * lax.rsqrt(jnp.maximum(msq, 1e-12))
        sim = lax.dot_general(
            xn, knorm_ref[...],
            dimension_numbers=(((1,), (1,)), ((), ())),
            preferred_element_type=jnp.float32,
        )
        sim_ref[...] = sim

        iota = lax.broadcasted_iota(jnp.int32, (B, P), 1)
        work = sim
        cols = []
        for _ in range(TOP_K):
            m = jnp.max(work, axis=1, keepdims=True)
            cand = jnp.where(work == m, iota, P)
            sel = jnp.min(cand, axis=1, keepdims=True)
            cols.append(sel)
            work = jnp.where(iota == sel, -1e30, work)
        idx_ref[...] = jnp.concatenate(cols, axis=1)


def _k1(x_embed, prompt_key):
    return pl.pallas_call(
        _k1_body,
        grid=(XSTEPS,),
        in_specs=[
            pl.BlockSpec((B, XBLK, D), lambda j: (0, j, 0)),
            pl.BlockSpec((P, D), lambda j: (0, 0)),
        ],
        out_specs=[
            pl.BlockSpec((B, XBLK, D), lambda j: (0, j, 0)),
            pl.BlockSpec((B, P), lambda j: (0, 0)),
            pl.BlockSpec((B, TOP_K), lambda j: (0, 0)),
        ],
        out_shape=[
            jax.ShapeDtypeStruct((B, OUT_S, D), jnp.float32),
            jax.ShapeDtypeStruct((B, P), jnp.float32),
            jax.ShapeDtypeStruct((B, TOP_K), jnp.int32),
        ],
        scratch_shapes=[
            pltpu.VMEM((B, D), jnp.float32),
            pltpu.VMEM((P, D), jnp.float32),
            pltpu.VMEM((B, PL, D), jnp.float32),
        ],
    )(x_embed, prompt_key)


def _kg_body(idx_ref, prompt_ref, prompted_in, out_ref, sem):
    del prompted_in
    copies = []
    for i in range(B * TOP_K):
        b, k = divmod(i, TOP_K)
        c = pltpu.make_async_copy(
            prompt_ref.at[idx_ref[i]],
            out_ref.at[b, pl.ds(k * L, L), :],
            sem,
        )
        c.start()
        copies.append(c)
    for c in copies:
        c.wait()


def _kg(idx_flat, prompt, prompted):
    grid_spec = pltpu.PrefetchScalarGridSpec(
        num_scalar_prefetch=1,
        grid=(1,),
        in_specs=[
            pl.BlockSpec(memory_space=pl.ANY),
            pl.BlockSpec(memory_space=pl.ANY),
        ],
        out_specs=pl.BlockSpec((B, PL, D), lambda i, idx: (0, 0, 0)),
        scratch_shapes=[
            pltpu.SemaphoreType.DMA,
        ],
    )
    return pl.pallas_call(
        _kg_body,
        grid_spec=grid_spec,
        out_shape=jax.ShapeDtypeStruct((B, OUT_S, D), jnp.float32),
        input_output_aliases={2: 0},
    )(idx_flat, prompt, prompted)


def kernel(x_embed, prompt, prompt_key):
    prompted, similarity, idx = _k1(x_embed, prompt_key)
    prompted = _kg(idx.reshape(B * TOP_K), prompt, prompted)
    return prompted, similarity, idx


# speedup vs baseline: 2.3543x; 1.0541x over previous
import jax
import jax.numpy as jnp
from jax import lax
from jax.experimental import pallas as pl
from jax.experimental.pallas import tpu as pltpu

B, S, D = 4, 4096, 1024
P, L = 1024, 16
TOP_K = 8
PL = TOP_K * L
OUT_S = PL + S
XBLK = 704
XSTEPS = OUT_S // XBLK


def _k1_body(x_ref, key_ref, out_ref, sim_ref, idx_ref, acc_ref, knorm_ref,
             carry_ref, ksem):
    j = pl.program_id(0)

    xb = x_ref[...]
    out_ref[:, 0:PL, :] = carry_ref[...]
    out_ref[:, PL:XBLK, :] = xb[:, 0 : XBLK - PL, :]
    carry_ref[...] = xb[:, XBLK - PL : XBLK, :]

    key_dma = pltpu.make_async_copy(key_ref, knorm_ref, ksem)

    @pl.when(j == 0)
    def _init():
        acc_ref[...] = jnp.sum(xb, axis=1)
        key_dma.start()

    @pl.when(j == 1)
    def _knorm():
        key_dma.wait()
        k = knorm_ref[...]
        ksq = jnp.sum(k * k, axis=1, keepdims=True)
        knorm_ref[...] = k * lax.rsqrt(jnp.maximum(ksq, 1e-12))

    @pl.when((j > 0) & (j < XSTEPS - 1))
    def _accum():
        acc_ref[...] += jnp.sum(xb, axis=1)

    @pl.when(j == XSTEPS - 1)
    def _accum_tail():
        acc_ref[...] += jnp.sum(xb[:, 0 : XBLK - PL, :], axis=1)

    @pl.when(j == XSTEPS - 1)
    def _finish():
        mean = acc_ref[...] * (1.0 / S)
        msq = jnp.sum(mean * mean, axis=1, keepdims=True)
        xn = mean * lax.rsqrt(jnp.maximum(msq, 1e-12))
        sim = lax.dot_general(
            xn, knorm_ref[...],
            dimension_numbers=(((1,), (1,)), ((), ())),
            preferred_element_type=jnp.float32,
        )
        sim_ref[...] = sim

        iota = lax.broadcasted_iota(jnp.int32, (B, P), 1)
        work = sim
        cols = []
        for _ in range(TOP_K):
            m = jnp.max(work, axis=1, keepdims=True)
            cand = jnp.where(work == m, iota, P)
            sel = jnp.min(cand, axis=1, keepdims=True)
            cols.append(sel)
            work = jnp.where(iota == sel, -1e30, work)
        idx_ref[...] = jnp.concatenate(cols, axis=1)


def _k1(x_embed, prompt_key):
    return pl.pallas_call(
        _k1_body,
        grid=(XSTEPS,),
        in_specs=[
            pl.BlockSpec((B, XBLK, D), lambda j: (0, j, 0)),
            pl.BlockSpec(memory_space=pl.ANY),
        ],
        out_specs=[
            pl.BlockSpec((B, XBLK, D), lambda j: (0, j, 0)),
            pl.BlockSpec((B, P), lambda j: (0, 0)),
            pl.BlockSpec((B, TOP_K), lambda j: (0, 0)),
        ],
        out_shape=[
            jax.ShapeDtypeStruct((B, OUT_S, D), jnp.float32),
            jax.ShapeDtypeStruct((B, P), jnp.float32),
            jax.ShapeDtypeStruct((B, TOP_K), jnp.int32),
        ],
        scratch_shapes=[
            pltpu.VMEM((B, D), jnp.float32),
            pltpu.VMEM((P, D), jnp.float32),
            pltpu.VMEM((B, PL, D), jnp.float32),
            pltpu.SemaphoreType.DMA,
        ],
        compiler_params=pltpu.CompilerParams(vmem_limit_bytes=63 * 1024 * 1024),
    )(x_embed, prompt_key)


def _kg_body(idx_ref, prompt_ref, prompted_in, out_ref, sem):
    del prompted_in
    copies = []
    for i in range(B * TOP_K):
        b, k = divmod(i, TOP_K)
        c = pltpu.make_async_copy(
            prompt_ref.at[idx_ref[i]],
            out_ref.at[b, pl.ds(k * L, L), :],
            sem,
        )
        c.start()
        copies.append(c)
    for c in copies:
        c.wait()


def _kg(idx_flat, prompt, prompted):
    grid_spec = pltpu.PrefetchScalarGridSpec(
        num_scalar_prefetch=1,
        grid=(1,),
        in_specs=[
            pl.BlockSpec(memory_space=pl.ANY),
            pl.BlockSpec(memory_space=pl.ANY),
        ],
        out_specs=pl.BlockSpec((B, PL, D), lambda i, idx: (0, 0, 0)),
        scratch_shapes=[
            pltpu.SemaphoreType.DMA,
        ],
    )
    return pl.pallas_call(
        _kg_body,
        grid_spec=grid_spec,
        out_shape=jax.ShapeDtypeStruct((B, OUT_S, D), jnp.float32),
        input_output_aliases={2: 0},
    )(idx_flat, prompt, prompted)


def kernel(x_embed, prompt, prompt_key):
    prompted, similarity, idx = _k1(x_embed, prompt_key)
    prompted = _kg(idx.reshape(B * TOP_K), prompt, prompted)
    return prompted, similarity, idx
